# baseline (speedup 1.0000x reference)
import jax
import jax.numpy as jnp
from jax import lax
from jax.experimental import pallas as pl
from jax.experimental.pallas import tpu as pltpu
from jax.experimental.pallas import tpu_sc as plsc

NC, NS, L = 2, 16, 16
B = 16384
HB = B // 2
V_DRV, V_WEEK, V_TIME = 24000, 7, 1440
D_DRV, D_WEEK, D_TIME = 16, 3, 8
D_OUT = D_DRV + D_WEEK + D_TIME + 1
UNROLL = 8
HGROUPS = HB // L


def _gather_loop(idx_v, tab_v, col_v):
  @plsc.parallel_loop(0, HGROUPS, step=1, unroll=UNROLL)
  def _(g):
    off = g * L
    iv = idx_v[pl.ds(off, L)]
    col_v[pl.ds(off, L)] = plsc.load_gather(tab_v, [iv])


def _retile(x_ref, o_ref):
  for c in range(D_OUT):
    o_ref[c, :] = x_ref[pl.ds(c * B, B)]


def _prep(wd_ref, wt_ref, ww_ref, od_ref, ot_ref, ow_ref):
  for c in range(D_DRV):
    od_ref[pl.ds(c * V_DRV, V_DRV)] = wd_ref[c, :]
  for c in range(D_TIME):
    ot_ref[pl.ds(c * V_TIME, V_TIME)] = wt_ref[c, :]
  for c in range(D_WEEK):
    ow_ref[pl.ds(c * 8, V_WEEK)] = ww_ref[c, :]


def _body(drv_hbm, wk_hbm, tm_hbm, dist_hbm, Wd_hbm, Ww_hbm, Wt_hbm,
          out_hbm, idx1_v, tab1_v, col1_v, idx2_v, tab2_v, col2_v,
          sem1, sem2, semw):
  wid = lax.axis_index("s") * NC + lax.axis_index("c")
  dcol = wid // 2
  dhalf = wid % 2
  doff = dhalf * HB

  pltpu.async_copy(Wd_hbm.at[pl.ds(dcol * V_DRV, V_DRV)], tab1_v, sem1)
  pltpu.async_copy(drv_hbm.at[pl.ds(doff, HB)], idx1_v, sem1)

  @pl.when(wid < 16)
  def _():
    pltpu.async_copy(Wt_hbm.at[pl.ds((wid // 2) * V_TIME, V_TIME)],
                     tab2_v.at[pl.ds(0, V_TIME)], sem2)
    pltpu.async_copy(tm_hbm.at[pl.ds(doff, HB)], idx2_v, sem2)

  @pl.when((wid >= 16) & (wid < 22))
  def _():
    w = wid - 16
    pltpu.async_copy(Ww_hbm.at[pl.ds((w // 2) * 8, V_WEEK)],
                     tab2_v.at[pl.ds(0, V_WEEK)], sem2)
    pltpu.async_copy(wk_hbm.at[pl.ds((w % 2) * HB, HB)], idx2_v, sem2)

  @pl.when((wid >= 22) & (wid < 24))
  def _():
    pltpu.async_copy(dist_hbm.at[pl.ds((wid - 22) * HB, HB)], col2_v, sem2)

  @pl.when(wid < 16)
  def _():
    pltpu.make_async_copy(Wt_hbm.at[pl.ds((wid // 2) * V_TIME, V_TIME)],
                          tab2_v.at[pl.ds(0, V_TIME)], sem2).wait()
    pltpu.make_async_copy(tm_hbm.at[pl.ds(doff, HB)], idx2_v, sem2).wait()
    _gather_loop(idx2_v, tab2_v, col2_v)
    dst = pl.ds((D_DRV + D_WEEK + wid // 2) * B + doff, HB)
    pltpu.async_copy(col2_v, out_hbm.at[dst], semw)

  @pl.when((wid >= 16) & (wid < 22))
  def _():
    w = wid - 16
    pltpu.make_async_copy(Ww_hbm.at[pl.ds((w // 2) * 8, V_WEEK)],
                          tab2_v.at[pl.ds(0, V_WEEK)], sem2).wait()
    pltpu.make_async_copy(wk_hbm.at[pl.ds((w % 2) * HB, HB)], idx2_v,
                          sem2).wait()
    _gather_loop(idx2_v, tab2_v, col2_v)
    dst = pl.ds((D_DRV + w // 2) * B + (w % 2) * HB, HB)
    pltpu.async_copy(col2_v, out_hbm.at[dst], semw)

  @pl.when((wid >= 22) & (wid < 24))
  def _():
    pltpu.make_async_copy(dist_hbm.at[pl.ds(0, HB)], col2_v, sem2).wait()
    dst = pl.ds((D_OUT - 1) * B + (wid - 22) * HB, HB)
    pltpu.async_copy(col2_v, out_hbm.at[dst], semw)

  pltpu.make_async_copy(Wd_hbm.at[pl.ds(dcol * V_DRV, V_DRV)], tab1_v,
                        sem1).wait()
  pltpu.make_async_copy(drv_hbm.at[pl.ds(doff, HB)], idx1_v, sem1).wait()
  _gather_loop(idx1_v, tab1_v, col1_v)
  pltpu.async_copy(col1_v, out_hbm.at[pl.ds(dcol * B + doff, HB)], semw)

  @pl.when(wid < 24)
  def _():
    pltpu.make_async_copy(col2_v, out_hbm.at[pl.ds(0, HB)], semw).wait()
  pltpu.make_async_copy(col1_v, out_hbm.at[pl.ds(dcol * B + doff, HB)],
                        semw).wait()


@jax.jit
def _run(driverID, weekID, timeID, dist, W_driver, W_week, W_time):
  wd_flat, wt_flat, ww_flat = pl.pallas_call(
      _prep,
      out_shape=[jax.ShapeDtypeStruct((D_DRV * V_DRV,), jnp.float32),
                 jax.ShapeDtypeStruct((D_TIME * V_TIME,), jnp.float32),
                 jax.ShapeDtypeStruct((D_WEEK * 8,), jnp.float32)],
  )(W_driver.T, W_time.T, W_week.T)
  mesh = plsc.VectorSubcoreMesh(core_axis_name="c", subcore_axis_name="s")
  out = pl.kernel(
      _body,
      out_type=jax.ShapeDtypeStruct((D_OUT * B,), jnp.float32),
      mesh=mesh,
      compiler_params=pltpu.CompilerParams(needs_layout_passes=False,
                                           use_tc_tiling_on_sc=False),
      scratch_types=[
          pltpu.VMEM((HB,), jnp.int32),
          pltpu.VMEM((V_DRV,), jnp.float32),
          pltpu.VMEM((HB,), jnp.float32),
          pltpu.VMEM((HB,), jnp.int32),
          pltpu.VMEM((V_TIME,), jnp.float32),
          pltpu.VMEM((HB,), jnp.float32),
          pltpu.SemaphoreType.DMA,
          pltpu.SemaphoreType.DMA,
          pltpu.SemaphoreType.DMA,
      ],
  )(driverID, weekID, timeID, dist, wd_flat, ww_flat, wt_flat)
  out2d = pl.pallas_call(
      _retile,
      out_shape=jax.ShapeDtypeStruct((D_OUT, B), jnp.float32))(out)
  return out2d.T


def kernel(driverID, weekID, timeID, dist, W_driver, W_week, W_time):
  return _run(driverID.astype(jnp.int32), weekID.astype(jnp.int32),
              timeID.astype(jnp.int32), dist.astype(jnp.float32),
              W_driver, W_week, W_time)


# speedup vs baseline: 1.0120x; 1.0120x over previous
import jax
import jax.numpy as jnp
from jax import lax
from jax.experimental import pallas as pl
from jax.experimental.pallas import tpu as pltpu
from jax.experimental.pallas import tpu_sc as plsc

NC, NS, L = 2, 16, 16
B = 16384
HB = B // 2
V_DRV, V_WEEK, V_TIME = 24000, 7, 1440
D_DRV, D_WEEK, D_TIME = 16, 3, 8
D_OUT = D_DRV + D_WEEK + D_TIME + 1
UNROLL = 8
HGROUPS = HB // L


def _gather_loop(idx_v, tab_v, col_v):
  @plsc.parallel_loop(0, HGROUPS, step=1, unroll=UNROLL)
  def _(g):
    off = g * L
    iv = idx_v[pl.ds(off, L)]
    col_v[pl.ds(off, L)] = plsc.load_gather(tab_v, [iv])


def _prep(wd_ref, wt_ref, ww_ref, od_ref, ot_ref, ow_ref):
  for c in range(D_DRV):
    od_ref[pl.ds(c * V_DRV, V_DRV)] = wd_ref[c, :]
  for c in range(D_TIME):
    ot_ref[pl.ds(c * V_TIME, V_TIME)] = wt_ref[c, :]
  for c in range(D_WEEK):
    ow_ref[pl.ds(c * 8, V_WEEK)] = ww_ref[c, :]


def _body(drv_hbm, wk_hbm, tm_hbm, dist_hbm, Wd_hbm, Ww_hbm, Wt_hbm,
          out_hbm, idx1_v, tab1_v, col1_v, idx2_v, tab2_v, col2_v,
          sem1, sem2, semw):
  wid = lax.axis_index("s") * NC + lax.axis_index("c")
  dcol = wid // 2
  dhalf = wid % 2
  doff = dhalf * HB

  pltpu.async_copy(Wd_hbm.at[pl.ds(dcol * V_DRV, V_DRV)], tab1_v, sem1)
  pltpu.async_copy(drv_hbm.at[pl.ds(doff, HB)], idx1_v, sem1)

  @pl.when(wid < 16)
  def _():
    pltpu.async_copy(Wt_hbm.at[pl.ds((wid // 2) * V_TIME, V_TIME)],
                     tab2_v.at[pl.ds(0, V_TIME)], sem2)
    pltpu.async_copy(tm_hbm.at[pl.ds(doff, HB)], idx2_v, sem2)

  @pl.when((wid >= 16) & (wid < 22))
  def _():
    w = wid - 16
    pltpu.async_copy(Ww_hbm.at[pl.ds((w // 2) * 8, V_WEEK)],
                     tab2_v.at[pl.ds(0, V_WEEK)], sem2)
    pltpu.async_copy(wk_hbm.at[pl.ds((w % 2) * HB, HB)], idx2_v, sem2)

  @pl.when((wid >= 22) & (wid < 24))
  def _():
    pltpu.async_copy(dist_hbm.at[pl.ds((wid - 22) * HB, HB)], col2_v, sem2)

  @pl.when(wid < 16)
  def _():
    pltpu.make_async_copy(Wt_hbm.at[pl.ds((wid // 2) * V_TIME, V_TIME)],
                          tab2_v.at[pl.ds(0, V_TIME)], sem2).wait()
    pltpu.make_async_copy(tm_hbm.at[pl.ds(doff, HB)], idx2_v, sem2).wait()
    _gather_loop(idx2_v, tab2_v, col2_v)
    dst = pl.ds((D_DRV + D_WEEK + wid // 2) * B + doff, HB)
    pltpu.async_copy(col2_v, out_hbm.at[dst], semw)

  @pl.when((wid >= 16) & (wid < 22))
  def _():
    w = wid - 16
    pltpu.make_async_copy(Ww_hbm.at[pl.ds((w // 2) * 8, V_WEEK)],
                          tab2_v.at[pl.ds(0, V_WEEK)], sem2).wait()
    pltpu.make_async_copy(wk_hbm.at[pl.ds((w % 2) * HB, HB)], idx2_v,
                          sem2).wait()
    _gather_loop(idx2_v, tab2_v, col2_v)
    dst = pl.ds((D_DRV + w // 2) * B + (w % 2) * HB, HB)
    pltpu.async_copy(col2_v, out_hbm.at[dst], semw)

  @pl.when((wid >= 22) & (wid < 24))
  def _():
    pltpu.make_async_copy(dist_hbm.at[pl.ds(0, HB)], col2_v, sem2).wait()
    dst = pl.ds((D_OUT - 1) * B + (wid - 22) * HB, HB)
    pltpu.async_copy(col2_v, out_hbm.at[dst], semw)

  pltpu.make_async_copy(Wd_hbm.at[pl.ds(dcol * V_DRV, V_DRV)], tab1_v,
                        sem1).wait()
  pltpu.make_async_copy(drv_hbm.at[pl.ds(doff, HB)], idx1_v, sem1).wait()
  _gather_loop(idx1_v, tab1_v, col1_v)
  pltpu.async_copy(col1_v, out_hbm.at[pl.ds(dcol * B + doff, HB)], semw)

  @pl.when(wid < 24)
  def _():
    pltpu.make_async_copy(col2_v, out_hbm.at[pl.ds(0, HB)], semw).wait()
  pltpu.make_async_copy(col1_v, out_hbm.at[pl.ds(dcol * B + doff, HB)],
                        semw).wait()


@jax.jit
def _run(driverID, weekID, timeID, dist, W_driver, W_week, W_time):
  wd_flat, wt_flat, ww_flat = pl.pallas_call(
      _prep,
      out_shape=[jax.ShapeDtypeStruct((D_DRV * V_DRV,), jnp.float32),
                 jax.ShapeDtypeStruct((D_TIME * V_TIME,), jnp.float32),
                 jax.ShapeDtypeStruct((D_WEEK * 8,), jnp.float32)],
  )(W_driver.T, W_time.T, W_week.T)
  mesh = plsc.VectorSubcoreMesh(core_axis_name="c", subcore_axis_name="s")
  out = pl.kernel(
      _body,
      out_type=jax.ShapeDtypeStruct((D_OUT * B,), jnp.float32),
      mesh=mesh,
      compiler_params=pltpu.CompilerParams(needs_layout_passes=False,
                                           use_tc_tiling_on_sc=False),
      scratch_types=[
          pltpu.VMEM((HB,), jnp.int32),
          pltpu.VMEM((V_DRV,), jnp.float32),
          pltpu.VMEM((HB,), jnp.float32),
          pltpu.VMEM((HB,), jnp.int32),
          pltpu.VMEM((V_TIME,), jnp.float32),
          pltpu.VMEM((HB,), jnp.float32),
          pltpu.SemaphoreType.DMA,
          pltpu.SemaphoreType.DMA,
          pltpu.SemaphoreType.DMA,
      ],
  )(driverID, weekID, timeID, dist, wd_flat, ww_flat, wt_flat)
  return out.reshape(D_OUT, B).T


def kernel(driverID, weekID, timeID, dist, W_driver, W_week, W_time):
  return _run(driverID.astype(jnp.int32), weekID.astype(jnp.int32),
              timeID.astype(jnp.int32), dist.astype(jnp.float32),
              W_driver, W_week, W_time)
